# Initial kernel scaffold; baseline (speedup 1.0000x reference)
import jax
import jax.numpy as jnp
from jax.experimental import pallas as pl


def kernel(state, edge_index, edge_index2, edge_index3, edge_index4, edge_index5, W1, b1, W2, b2, W3, b3, lW1, lb1, lW2, lb2, lW3, lb3):
    raise NotImplementedError("write your pallas kernel here")


# speedup vs baseline: 12.5401x; 12.5401x over previous
import functools

import jax
import jax.numpy as jnp
from jax import lax
from jax.experimental import pallas as pl
from jax.experimental.pallas import tpu as pltpu
from jax.experimental.pallas import tpu_sc as plsc

N = 10000
E = 320000
D = 128
H = 32

NC = 2
NS = 16
NW = NC * NS
CHUNK = 128
CH = 79
SLOTS = CH * CHUNK
EPAD = NW * SLOTS
PAD = EPAD - E
NA = 10240
SPAN_A = NA // NS
NROWS = 10240
SPAN_C = NROWS // NS
BLK = 400

_sc_mesh = plsc.VectorSubcoreMesh(core_axis_name="c", subcore_axis_name="s")


@functools.partial(
    pl.kernel,
    mesh=_sc_mesh,
    out_type=jax.ShapeDtypeStruct((NC * 5 * NA,), jnp.float32),
    compiler_params=pltpu.CompilerParams(needs_layout_passes=False),
    scratch_types=[
        pltpu.VMEM((1, SLOTS), jnp.int32),
        pltpu.VMEM((NA,), jnp.float32),
        pltpu.VMEM_SHARED((NS * NA,), jnp.float32),
        pltpu.VMEM((NS * SPAN_A,), jnp.float32),
        pltpu.VMEM((SPAN_A,), jnp.float32),
    ],
)
def _deg_kernel(d1, d2, d3, d4, d5, out, idx_v, deg_v, stage, red_v, res_v):
    cid = lax.axis_index("c")
    sid = lax.axis_index("s")
    wid = cid * NS + sid
    ones = jnp.ones((16,), jnp.float32)
    for k, dref in enumerate((d1, d2, d3, d4, d5)):
        pltpu.sync_copy(dref.at[pl.ds(wid, 1)], idx_v)

        def _zero(i, c):
            deg_v[pl.ds(i * 16, 16)] = jnp.zeros((16,), jnp.float32)
            return c
        lax.fori_loop(0, NA // 16, _zero, 0)

        def _scat(e, c):
            idx = idx_v[0, pl.ds(e * 16, 16)]
            plsc.addupdate_scatter(deg_v, [idx], ones)
            return c
        lax.fori_loop(0, SLOTS // 16, _scat, 0)

        pltpu.sync_copy(deg_v, stage.at[pl.ds(sid * NA, NA)])
        plsc.subcore_barrier()
        for r in range(NS):
            pltpu.sync_copy(stage.at[pl.ds(r * NA + sid * SPAN_A, SPAN_A)],
                            red_v.at[pl.ds(r * SPAN_A, SPAN_A)])

        def _red(i, c):
            acc = jnp.zeros((16,), jnp.float32)
            for r in range(NS):
                acc = acc + red_v[pl.ds(r * SPAN_A + i * 16, 16)]
            res_v[pl.ds(i * 16, 16)] = acc
            return c
        lax.fori_loop(0, SPAN_A // 16, _red, 0)
        pltpu.sync_copy(res_v, out.at[pl.ds(cid * (5 * NA) + k * NA
                                            + sid * SPAN_A, SPAN_A)])
        plsc.subcore_barrier()


@functools.partial(
    pl.kernel,
    mesh=_sc_mesh,
    out_type=jax.ShapeDtypeStruct((NC, 5, NROWS, D), jnp.float32),
    scratch_types=[
        pltpu.VMEM((CH, CHUNK), jnp.int32),
        pltpu.VMEM((CH, CHUNK), jnp.int32),
        pltpu.VMEM((CHUNK, D), jnp.float32),
        pltpu.VMEM_SHARED((NROWS, D), jnp.float32),
    ],
)
def _agg_kernel(s1, d1, s2, d2, s3, d3, s4, d4, s5, d5, z_hbm, hs1, hs2, hs3,
                hs4, hs5, out, src_v, dst_v, rows_v, acc):
    cid = lax.axis_index("c")
    sid = lax.axis_index("s")
    wid = cid * NS + sid
    sets = ((s1, d1, hs1), (s2, d2, hs2), (s3, d3, hs3), (s4, d4, hs4),
            (s5, d5, hs5))
    for k, (sref, dref, hsref) in enumerate(sets):
        pltpu.sync_copy(sref.at[wid], src_v)
        pltpu.sync_copy(dref.at[wid], dst_v)
        pltpu.sync_copy(z_hbm, acc.at[pl.ds(sid * SPAN_C, SPAN_C)])
        plsc.subcore_barrier()

        def _edge(j, c):
            pltpu.sync_copy(hsref.at[src_v.at[j]], rows_v)
            pltpu.sync_copy(rows_v, acc.at[dst_v.at[j]], add=True)
            return c
        lax.fori_loop(0, CH, _edge, 0)
        plsc.subcore_barrier()
        pltpu.sync_copy(acc.at[pl.ds(sid * SPAN_C, SPAN_C)],
                        out.at[cid, k, pl.ds(sid * SPAN_C, SPAN_C)])
        plsc.subcore_barrier()


def _feat_body(x_ref, w1_ref, w2_ref, w3_ref, deg_ref, o1, o2, o3, o4, o5):
    x = x_ref[...]
    dp = deg_ref[...]
    deg = dp[:, 0, :] + dp[:, 1, :]
    dinv = jnp.where(deg > 0, lax.rsqrt(deg), 0.0)
    h1 = jnp.dot(x, w1_ref[...], preferred_element_type=jnp.float32)
    h2 = jnp.dot(x, w2_ref[...], preferred_element_type=jnp.float32)
    h3 = jnp.dot(x, w3_ref[...], preferred_element_type=jnp.float32)
    o1[...] = h1 * dinv[:, 0:1]
    o2[...] = h2 * dinv[:, 1:2]
    o3[...] = h3 * dinv[:, 2:3]
    o4[...] = h3 * dinv[:, 3:4]
    o5[...] = h3 * dinv[:, 4:5]


def _head_body(parts_ref, x_ref, deg_ref, bias_ref, lw1_ref, lb1_ref, lw2_ref,
               lb2_ref, lw3_ref, lb3_ref, conc_ref, tot_ref):
    i = pl.program_id(0)
    dp = deg_ref[...]
    deg = dp[:, 0, :] + dp[:, 1, :]
    dinv = jnp.where(deg > 0, lax.rsqrt(deg), 0.0)
    z = jnp.dot(x_ref[...], lw1_ref[pl.ds(5 * D, D), :],
                preferred_element_type=jnp.float32)
    for k in range(5):
        y = (parts_ref[0, k] + parts_ref[1, k]) * dinv[:, k:k + 1]
        y = y + bias_ref[k:k + 1, :]
        o = jnp.maximum(y, 0.0)
        z = z + jnp.dot(o, lw1_ref[pl.ds(k * D, D), :],
                        preferred_element_type=jnp.float32)
    z = z + lb1_ref[...]
    z = jnp.where(z >= 0, z, 0.01 * z)
    z = jnp.dot(z, lw2_ref[...], preferred_element_type=jnp.float32)
    z = z + lb2_ref[...]
    z = jnp.where(z >= 0, z, 0.01 * z)
    u = jnp.dot(z, lw3_ref[...], preferred_element_type=jnp.float32)
    u = u + lb3_ref[...]
    c = jax.nn.softplus(u)
    conc_ref[...] = c

    @pl.when(i == 0)
    def _init():
        tot_ref[...] = jnp.zeros((1, 1), jnp.float32)

    tot_ref[...] = tot_ref[...] + jnp.sum(c).reshape(1, 1)


def _norm_body(c_ref, tot_ref, act_ref, reg_ref):
    t = tot_ref[...]
    act_ref[...] = c_ref[...] / (t + 1e-20)
    reg_ref[...] = t / float(N)


def kernel(state, edge_index, edge_index2, edge_index3, edge_index4,
           edge_index5, W1, b1, W2, b2, W3, b3,
           lW1, lb1, lW2, lb2, lW3, lb3):
    f32 = jnp.float32

    def prep(e):
        src = jnp.concatenate([e[0], jnp.zeros((PAD,), jnp.int32)])
        dst = jnp.concatenate([e[1], jnp.full((PAD,), N, jnp.int32)])
        return (src.reshape(NW, CH, CHUNK), dst.reshape(NW, CH, CHUNK),
                dst.reshape(NW, SLOTS))

    s1, d1, dw1 = prep(edge_index)
    s2, d2, dw2 = prep(edge_index2)
    s3, d3, dw3 = prep(edge_index3)
    s4, d4, dw4 = prep(edge_index4)
    s5, d5, dw5 = prep(edge_index5)

    deg_parts = _deg_kernel(dw1, dw2, dw3, dw4, dw5).reshape(NC, 5, NA)
    degT = jnp.transpose(deg_parts, (2, 0, 1))[:N]

    hs = pl.pallas_call(
        _feat_body,
        grid=(N // BLK,),
        in_specs=[
            pl.BlockSpec((BLK, D), lambda i: (i, 0)),
            pl.BlockSpec((D, D), lambda i: (0, 0)),
            pl.BlockSpec((D, D), lambda i: (0, 0)),
            pl.BlockSpec((D, D), lambda i: (0, 0)),
            pl.BlockSpec((BLK, NC, 5), lambda i: (i, 0, 0)),
        ],
        out_specs=[pl.BlockSpec((BLK, D), lambda i: (i, 0))] * 5,
        out_shape=[jax.ShapeDtypeStruct((N, D), f32)] * 5,
    )(state, W1, W2, W3, degT)

    zeros_hbm = jnp.zeros((SPAN_C, D), f32)
    parts = _agg_kernel(s1, d1, s2, d2, s3, d3, s4, d4, s5, d5, zeros_hbm,
                        hs[0], hs[1], hs[2], hs[3], hs[4])

    bias_mat = jnp.stack([b1, b2, b3, b3, b3])
    lb1_2 = lb1.reshape(1, H)
    lb2_2 = lb2.reshape(1, H)
    lb3_2 = lb3.reshape(1, 1)

    conc, tot = pl.pallas_call(
        _head_body,
        grid=(N // BLK,),
        in_specs=[
            pl.BlockSpec((NC, 5, BLK, D), lambda i: (0, 0, i, 0)),
            pl.BlockSpec((BLK, D), lambda i: (i, 0)),
            pl.BlockSpec((BLK, NC, 5), lambda i: (i, 0, 0)),
            pl.BlockSpec((5, D), lambda i: (0, 0)),
            pl.BlockSpec((6 * D, H), lambda i: (0, 0)),
            pl.BlockSpec((1, H), lambda i: (0, 0)),
            pl.BlockSpec((H, H), lambda i: (0, 0)),
            pl.BlockSpec((1, H), lambda i: (0, 0)),
            pl.BlockSpec((H, 1), lambda i: (0, 0)),
            pl.BlockSpec((1, 1), lambda i: (0, 0)),
        ],
        out_specs=[
            pl.BlockSpec((BLK, 1), lambda i: (i, 0)),
            pl.BlockSpec((1, 1), lambda i: (0, 0)),
        ],
        out_shape=[
            jax.ShapeDtypeStruct((N, 1), f32),
            jax.ShapeDtypeStruct((1, 1), f32),
        ],
    )(parts, state, degT, bias_mat, lW1, lb1_2, lW2, lb2_2, lW3, lb3_2)

    action2d, reg2d = pl.pallas_call(
        _norm_body,
        grid=(N // BLK,),
        in_specs=[
            pl.BlockSpec((BLK, 1), lambda i: (i, 0)),
            pl.BlockSpec((1, 1), lambda i: (0, 0)),
        ],
        out_specs=[
            pl.BlockSpec((BLK, 1), lambda i: (i, 0)),
            pl.BlockSpec((1, 1), lambda i: (0, 0)),
        ],
        out_shape=[
            jax.ShapeDtypeStruct((N, 1), f32),
            jax.ShapeDtypeStruct((1, 1), f32),
        ],
    )(conc, tot)

    return (action2d.reshape(N), reg2d[0, 0])
